# baseline (speedup 1.0000x reference)
import jax
import jax.numpy as jnp
from jax import lax
from jax.experimental import pallas as pl
from jax.experimental.pallas import tpu as pltpu
from jax.experimental.pallas import tpu_sc as plsc

_F = 26
_V = 100000
_D = 16
_B = 16384

_NC = 2
_NS = 16
_NW = _NC * _NS
_CH = 128
_JPW = _B // _NW // _CH
_BPW = _JPW * _CH
_NCH = _F * _JPW
_NBUF = 8
_AHEAD = 4
_NROUND = _NCH // _NBUF


def _body(idx_hbm, tab_hbm, out_hbm, idx_v, rows, gsem, osem):
    c = lax.axis_index("c")
    s = lax.axis_index("s")
    wid = s * _NC + c
    b0 = wid * _BPW

    pltpu.sync_copy(idx_hbm.at[:, pl.ds(wid * _JPW, _JPW)], idx_v)

    def gather(q, b):
        f = q // _JPW
        j = lax.rem(q, _JPW)
        return pltpu.make_async_copy(
            tab_hbm.at[idx_v.at[f, j]], rows.at[b], gsem[b])

    def ocopy(q, b):
        f = q // _JPW
        j = lax.rem(q, _JPW)
        return pltpu.make_async_copy(
            rows.at[b], out_hbm.at[pl.ds(b0 + j * _CH, _CH), f], osem[b])

    for p in range(_AHEAD):
        gather(p, p).start()

    def round_(r, carry):
        q0 = r * _NBUF
        for b in range(_NBUF):
            q = q0 + b
            p = q + _AHEAD
            bp = (b + _AHEAD) % _NBUF

            @pl.when((p >= _NBUF) & (p < _NCH))
            def _free_buf():
                ocopy(p - _NBUF, bp).wait()

            @pl.when(p < _NCH)
            def _start_next():
                gather(p, bp).start()

            gather(q, b).wait()
            ocopy(q, b).start()
        return carry

    lax.fori_loop(0, _NROUND, round_, 0)

    for b in range(_NBUF):
        ocopy(_NCH - _NBUF + b, b).wait()


@jax.jit
def kernel(indices, tables):
    flat_idx = indices.astype(jnp.int32).T + (
        jnp.arange(_F, dtype=jnp.int32) * _V
    )[:, None]
    flat_idx = flat_idx.reshape(_F, _B // _CH, _CH)
    tab_flat = tables.reshape(_F * _V, _D)

    mesh = plsc.VectorSubcoreMesh(
        core_axis_name="c", subcore_axis_name="s",
        num_cores=_NC, num_subcores=_NS,
    )
    return pl.kernel(
        _body,
        out_type=jax.ShapeDtypeStruct((_B, _F, _D), jnp.float32),
        mesh=mesh,
        scratch_types=[
            pltpu.VMEM((_F, _JPW, _CH), jnp.int32),
            pltpu.VMEM((_NBUF, _CH, _D), jnp.float32),
            [pltpu.SemaphoreType.DMA] * _NBUF,
            [pltpu.SemaphoreType.DMA] * _NBUF,
        ],
        compiler_params=pltpu.CompilerParams(use_tc_tiling_on_sc=False),
    )(flat_idx, tab_flat)
